# baseline (speedup 1.0000x reference)
import functools

import jax
import jax.numpy as jnp
from jax import lax
from jax.experimental import pallas as pl
from jax.experimental.pallas import tpu as pltpu
from jax.experimental.pallas import tpu_sc as plsc

_NC = 2
_NS = 16
_NW = _NC * _NS
_LANES = 16
_R = 16
_NB = 3


def _make_sc_add(B, T, C):
    rows_per_w = T // _NW
    nchunk = rows_per_w // _R
    nj = nchunk * B
    groups = C // _LANES
    mesh = plsc.VectorSubcoreMesh(core_axis_name="c", subcore_axis_name="s")

    @functools.partial(
        pl.kernel,
        mesh=mesh,
        out_type=jax.ShapeDtypeStruct((B, T, C), jnp.float32),
        scratch_types=[
            pltpu.VMEM((_NB, _R, C), jnp.float32),
            pltpu.VMEM((2, _R, C), jnp.float32),
            pltpu.SemaphoreType.DMA((_NB,)),
            pltpu.SemaphoreType.DMA((_NB,)),
            pltpu.SemaphoreType.DMA((2,)),
        ],
    )
    def sc_add(x_hbm, pos_hbm, out_hbm, xb, pb, sin, sout, spos):
        wid = lax.axis_index("s") * _NC + lax.axis_index("c")
        base = wid * rows_per_w

        def in_copy(j, slot):
            c = j // B
            b = j % B
            row0 = base + c * _R
            return pltpu.make_async_copy(
                x_hbm.at[b, pl.ds(row0, _R), :], xb.at[slot], sin.at[slot])

        def out_copy(j, slot):
            c = j // B
            b = j % B
            row0 = base + c * _R
            return pltpu.make_async_copy(
                xb.at[slot], out_hbm.at[b, pl.ds(row0, _R), :], sout.at[slot])

        def pos_copy(c, pslot):
            row0 = base + c * _R
            return pltpu.make_async_copy(
                pos_hbm.at[pl.ds(row0, _R), :], pb.at[pslot], spos.at[pslot])

        in_copy(0, 0).start()
        pos_copy(0, 0).start()
        in_copy(1, 1).start()

        def job(j, carry):
            slot = j % _NB
            c = j // B
            b = j % B
            pslot = c % 2

            @pl.when(j >= 1)
            def _():
                out_copy(j - 1, (j - 1) % _NB).wait()

            @pl.when(j + 2 < nj)
            def _():
                in_copy(j + 2, (j + 2) % _NB).start()

            @pl.when(jnp.logical_and(b == 0, c + 1 < nchunk))
            def _():
                pos_copy(c + 1, (c + 1) % 2).start()

            in_copy(j, slot).wait()

            @pl.when(b == 0)
            def _():
                pos_copy(c, pslot).wait()

            def row(r, rc):
                for g in range(groups):
                    sl = pl.ds(g * _LANES, _LANES)
                    xb[slot, r, sl] = xb[slot, r, sl] + pb[pslot, r, sl]
                return rc

            lax.fori_loop(0, _R, row, 0)
            out_copy(j, slot).start()
            return carry

        lax.fori_loop(0, nj, job, 0)
        out_copy(nj - 1, (nj - 1) % _NB).wait()

    return sc_add


def kernel(x, pos_emb):
    B, T, C = x.shape
    return _make_sc_add(B, T, C)(x, pos_emb)


# speedup vs baseline: 3.9603x; 3.9603x over previous
import jax
import jax.numpy as jnp
from jax.experimental import pallas as pl


def _add_body(x_ref, p_ref, o_ref):
    o_ref[...] = x_ref[...] + p_ref[...]


def kernel(x, pos_emb):
    B, T, C = x.shape
    BT = 2048
    grid = (T // BT, B)
    return pl.pallas_call(
        _add_body,
        grid=grid,
        in_specs=[
            pl.BlockSpec((1, BT, C), lambda i, j: (j, i, 0)),
            pl.BlockSpec((BT, C), lambda i, j: (i, 0)),
        ],
        out_specs=pl.BlockSpec((1, BT, C), lambda i, j: (j, i, 0)),
        out_shape=jax.ShapeDtypeStruct((B, T, C), x.dtype),
    )(x, pos_emb)
